# baseline (speedup 1.0000x reference)
import functools

import jax
import jax.numpy as jnp
from jax import lax
from jax.experimental import pallas as pl
from jax.experimental.pallas import tpu as pltpu
from jax.experimental.pallas import tpu_sc as plsc

N = 10000
E = 320000
D = 128
NC = 2
NS = 16
NW = NC * NS
EW = E // NW
EP = 327680
EWP = EP // NW
CH = 40
NCHUNK = EWP // CH
IBLK = 64
NBLK = NCHUNK // IBLK
NP = 10240
RW = NP // NS

BN = 2000
GRID = N // BN

_mesh = plsc.VectorSubcoreMesh(core_axis_name="c", subcore_axis_name="s")


@functools.partial(
    pl.kernel,
    out_type=jax.ShapeDtypeStruct((NW * NP,), jnp.float32),
    mesh=_mesh,
    scratch_types=[
        pltpu.VMEM((EW,), jnp.int32),
        pltpu.VMEM((NP,), jnp.float32),
    ],
    compiler_params=pltpu.CompilerParams(needs_layout_passes=False),
)
def _sc_degree(dst_hbm, zeros_hbm, out_hbm, dstv, degt):
    c = lax.axis_index("c")
    s = lax.axis_index("s")
    wid = c * NS + s
    pltpu.sync_copy(dst_hbm.at[pl.ds(wid * EW, EW)], dstv)
    pltpu.sync_copy(zeros_hbm, degt)
    ones = jnp.ones((16,), jnp.float32)

    def grp(g, carry):
        idx = dstv[pl.ds(g * 16, 16)]
        plsc.addupdate_scatter(degt, [idx], ones)
        return carry

    lax.fori_loop(0, EW // 16, grp, 0)
    pltpu.sync_copy(degt, out_hbm.at[pl.ds(wid * NP, NP)])


@functools.partial(
    pl.kernel,
    out_type=jax.ShapeDtypeStruct((NC * NP, D), jnp.float32),
    mesh=_mesh,
    scratch_types=[
        pltpu.VMEM((IBLK, CH), jnp.int32),
        pltpu.VMEM((IBLK, CH), jnp.int32),
        pltpu.VMEM((CH, D), jnp.float32),
        pltpu.VMEM((CH, D), jnp.float32),
        pltpu.SemaphoreType.DMA,
        pltpu.SemaphoreType.DMA,
        pltpu.VMEM_SHARED((NP, D), jnp.float32),
    ],
)
def _sc_edge_agg(hs_hbm, src_hbm, dst_hbm, zeros_hbm, out_hbm,
                 srcv, dstv, rows0, rows1, sem0, sem1, acc):
    c = lax.axis_index("c")
    s = lax.axis_index("s")
    wid = c * NS + s
    pltpu.sync_copy(zeros_hbm, acc.at[pl.ds(s * RW, RW)])
    plsc.subcore_barrier()

    def block(b, carry):
        pltpu.sync_copy(src_hbm.at[wid].at[b], srcv)
        pltpu.sync_copy(dst_hbm.at[wid].at[b], dstv)
        pltpu.async_copy(hs_hbm.at[srcv.at[0]], rows0, sem0)
        pltpu.async_copy(hs_hbm.at[srcv.at[1]], rows1, sem1)

        def pair(i, c2):
            j0 = 2 * i
            pltpu.make_async_copy(hs_hbm.at[srcv.at[j0]], rows0, sem0).wait()
            pltpu.sync_copy(rows0, acc.at[dstv.at[j0]], add=True)
            pltpu.async_copy(hs_hbm.at[srcv.at[j0 + 2]], rows0, sem0)
            pltpu.make_async_copy(hs_hbm.at[srcv.at[j0 + 1]], rows1,
                                  sem1).wait()
            pltpu.sync_copy(rows1, acc.at[dstv.at[j0 + 1]], add=True)
            pltpu.async_copy(hs_hbm.at[srcv.at[j0 + 3]], rows1, sem1)
            return c2

        lax.fori_loop(0, IBLK // 2 - 1, pair, 0)
        pltpu.make_async_copy(hs_hbm.at[srcv.at[IBLK - 2]], rows0,
                              sem0).wait()
        pltpu.sync_copy(rows0, acc.at[dstv.at[IBLK - 2]], add=True)
        pltpu.make_async_copy(hs_hbm.at[srcv.at[IBLK - 1]], rows1,
                              sem1).wait()
        pltpu.sync_copy(rows1, acc.at[dstv.at[IBLK - 1]], add=True)
        return carry

    lax.fori_loop(0, NBLK, block, 0)
    plsc.subcore_barrier()
    pltpu.sync_copy(acc.at[pl.ds(s * RW, RW)],
                    out_hbm.at[pl.ds(c * NP + s * RW, RW)])


def _tc_first(x_ref, w_ref, deg_ref, hs_ref, dinv_ref):
    deg = jnp.sum(deg_ref[...], axis=1, keepdims=True) + 1.0
    dinv = lax.rsqrt(deg)
    h = jnp.dot(x_ref[...], w_ref[...], preferred_element_type=jnp.float32)
    hs_ref[...] = h * dinv
    dinv_ref[...] = dinv


def _tc_mid(agg_ref, hs1_ref, dinv_ref, b_ref, w_ref, hs2_ref):
    a = agg_ref[...]
    dinv = dinv_ref[...]
    z = (a[0] + a[1] + hs1_ref[...]) * dinv + b_ref[...]
    o = jnp.maximum(z, 0.0)
    hs2_ref[...] = jnp.dot(o, w_ref[...],
                           preferred_element_type=jnp.float32) * dinv


def _tc_last(agg_ref, hs2_ref, dinv_ref, b_ref, fcw_ref, fcb_ref,
             h_ref, s_ref):
    a = agg_ref[...]
    z = (a[0] + a[1] + hs2_ref[...]) * dinv_ref[...] + b_ref[...]
    h = jnp.maximum(z, 0.0)
    h_ref[...] = h
    s_ref[...] = jnp.dot(h, fcw_ref[...],
                         preferred_element_type=jnp.float32) + fcb_ref[...]


_first = pl.pallas_call(
    _tc_first,
    grid=(GRID,),
    in_specs=[
        pl.BlockSpec((BN, D), lambda i: (i, 0)),
        pl.BlockSpec((D, D), lambda i: (0, 0)),
        pl.BlockSpec((BN, NW), lambda i: (i, 0)),
    ],
    out_specs=[
        pl.BlockSpec((BN, D), lambda i: (i, 0)),
        pl.BlockSpec((BN, 1), lambda i: (i, 0)),
    ],
    out_shape=[
        jax.ShapeDtypeStruct((N, D), jnp.float32),
        jax.ShapeDtypeStruct((N, 1), jnp.float32),
    ],
)

_mid = pl.pallas_call(
    _tc_mid,
    grid=(GRID,),
    in_specs=[
        pl.BlockSpec((NC, BN, D), lambda i: (0, i, 0)),
        pl.BlockSpec((BN, D), lambda i: (i, 0)),
        pl.BlockSpec((BN, 1), lambda i: (i, 0)),
        pl.BlockSpec((1, D), lambda i: (0, 0)),
        pl.BlockSpec((D, D), lambda i: (0, 0)),
    ],
    out_specs=pl.BlockSpec((BN, D), lambda i: (i, 0)),
    out_shape=jax.ShapeDtypeStruct((N, D), jnp.float32),
)

_last = pl.pallas_call(
    _tc_last,
    grid=(GRID,),
    in_specs=[
        pl.BlockSpec((NC, BN, D), lambda i: (0, i, 0)),
        pl.BlockSpec((BN, D), lambda i: (i, 0)),
        pl.BlockSpec((BN, 1), lambda i: (i, 0)),
        pl.BlockSpec((1, D), lambda i: (0, 0)),
        pl.BlockSpec((D, 1), lambda i: (0, 0)),
        pl.BlockSpec((1, 1), lambda i: (0, 0)),
    ],
    out_specs=[
        pl.BlockSpec((BN, D), lambda i: (i, 0)),
        pl.BlockSpec((BN, 1), lambda i: (i, 0)),
    ],
    out_shape=[
        jax.ShapeDtypeStruct((N, D), jnp.float32),
        jax.ShapeDtypeStruct((N, 1), jnp.float32),
    ],
)


def kernel(x, edge_index, W1, b1, W2, b2, fc_w, fc_b):
    pad_src = jnp.zeros((EP - E,), jnp.int32)
    pad_dst = N + jnp.arange(EP - E, dtype=jnp.int32) % (NP - N)
    src3 = jnp.concatenate([edge_index[0], pad_src]).reshape(
        NW, NBLK, IBLK, CH)
    dst3 = jnp.concatenate([edge_index[1], pad_dst]).reshape(
        NW, NBLK, IBLK, CH)
    zeros_rowsD = jnp.zeros((RW, D), jnp.float32)
    zeros_np = jnp.zeros((NP,), jnp.float32)

    degp = _sc_degree(edge_index[1], zeros_np)
    degp = degp.reshape(NW, NP)[:, :N].T
    hs1, dinv = _first(x, W1, degp)
    agg1 = _sc_edge_agg(hs1, src3, dst3, zeros_rowsD).reshape(NC, NP, D)
    hs2 = _mid(agg1, hs1, dinv, b1.reshape(1, D), W2)
    agg2 = _sc_edge_agg(hs2, src3, dst3, zeros_rowsD).reshape(NC, NP, D)
    h, s = _last(agg2, hs2, dinv, b2.reshape(1, D), fc_w,
                 fc_b.reshape(1, 1))
    return s[:, 0], h


# speedup vs baseline: 2.2230x; 2.2230x over previous
import functools

import jax
import jax.numpy as jnp
from jax import lax
from jax.experimental import pallas as pl
from jax.experimental.pallas import tpu as pltpu
from jax.experimental.pallas import tpu_sc as plsc

N = 10000
E = 320000
D = 128
NC = 2
NS = 16
NW = NC * NS
EW = E // NW
CH = 40
NCHUNK = EW // CH
IBLK = 50
NBLK = NCHUNK // IBLK
NP = 10240
RW = NP // NS
DEGW = 16

BN = 2000
GRID = N // BN

_mesh = plsc.VectorSubcoreMesh(core_axis_name="c", subcore_axis_name="s")


@functools.partial(
    pl.kernel,
    out_type=jax.ShapeDtypeStruct((NW * NP,), jnp.float32),
    mesh=_mesh,
    scratch_types=[
        pltpu.VMEM((EW,), jnp.int32),
        pltpu.VMEM((NP,), jnp.float32),
    ],
    compiler_params=pltpu.CompilerParams(needs_layout_passes=False),
)
def _sc_degree(dst_hbm, zeros_hbm, out_hbm, dstv, degt):
    c = lax.axis_index("c")
    s = lax.axis_index("s")
    wid = c * NS + s
    pltpu.sync_copy(dst_hbm.at[pl.ds(wid * EW, EW)], dstv)
    pltpu.sync_copy(zeros_hbm, degt)
    ones = jnp.ones((16,), jnp.float32)

    def grp(g, carry):
        idx = dstv[pl.ds(g * 16, 16)]
        plsc.addupdate_scatter(degt, [idx], ones)
        return carry

    lax.fori_loop(0, EW // 16, grp, 0)
    pltpu.sync_copy(degt, out_hbm.at[pl.ds(wid * NP, NP)])


@functools.partial(
    pl.kernel,
    out_type=jax.ShapeDtypeStruct((NC * NP, D), jnp.float32),
    mesh=_mesh,
    scratch_types=[
        pltpu.VMEM((IBLK, CH), jnp.int32),
        pltpu.VMEM((IBLK, CH), jnp.int32),
        pltpu.VMEM((CH, D), jnp.float32),
        pltpu.VMEM((CH, D), jnp.float32),
        pltpu.SemaphoreType.DMA,
        pltpu.SemaphoreType.DMA,
        pltpu.SemaphoreType.DMA,
        pltpu.SemaphoreType.DMA,
        pltpu.VMEM_SHARED((NP, D), jnp.float32),
    ],
)
def _sc_edge_agg(hs_hbm, src_hbm, dst_hbm, zeros_hbm, out_hbm,
                 srcv, dstv, rows0, rows1, sem0, sem1, sem0s, sem1s, acc):
    c = lax.axis_index("c")
    s = lax.axis_index("s")
    wid = c * NS + s
    pltpu.sync_copy(zeros_hbm, acc.at[pl.ds(s * RW, RW)])
    plsc.subcore_barrier()

    def block(b, carry):
        pltpu.sync_copy(src_hbm.at[wid].at[b], srcv)
        pltpu.sync_copy(dst_hbm.at[wid].at[b], dstv)
        pltpu.async_copy(hs_hbm.at[srcv.at[0]], rows0, sem0)
        pltpu.async_copy(hs_hbm.at[srcv.at[1]], rows1, sem1)

        def pair(i, c2):
            j0 = 2 * i
            pltpu.make_async_copy(hs_hbm.at[srcv.at[j0]], rows0, sem0).wait()
            pltpu.async_copy(rows0, acc.at[dstv.at[j0]], sem0s, add=True)
            pltpu.make_async_copy(hs_hbm.at[srcv.at[j0 + 1]], rows1,
                                  sem1).wait()
            pltpu.async_copy(rows1, acc.at[dstv.at[j0 + 1]], sem1s, add=True)
            pltpu.make_async_copy(rows0, acc.at[dstv.at[j0]], sem0s).wait()
            pltpu.async_copy(hs_hbm.at[srcv.at[j0 + 2]], rows0, sem0)
            pltpu.make_async_copy(rows1, acc.at[dstv.at[j0 + 1]],
                                  sem1s).wait()
            pltpu.async_copy(hs_hbm.at[srcv.at[j0 + 3]], rows1, sem1)
            return c2

        lax.fori_loop(0, IBLK // 2 - 1, pair, 0)
        pltpu.make_async_copy(hs_hbm.at[srcv.at[IBLK - 2]], rows0,
                              sem0).wait()
        pltpu.async_copy(rows0, acc.at[dstv.at[IBLK - 2]], sem0s, add=True)
        pltpu.make_async_copy(hs_hbm.at[srcv.at[IBLK - 1]], rows1,
                              sem1).wait()
        pltpu.async_copy(rows1, acc.at[dstv.at[IBLK - 1]], sem1s, add=True)
        pltpu.make_async_copy(rows0, acc.at[dstv.at[IBLK - 2]],
                              sem0s).wait()
        pltpu.make_async_copy(rows1, acc.at[dstv.at[IBLK - 1]],
                              sem1s).wait()
        return carry

    lax.fori_loop(0, NBLK, block, 0)
    plsc.subcore_barrier()
    pltpu.sync_copy(acc.at[pl.ds(s * RW, RW)],
                    out_hbm.at[pl.ds(c * NP + s * RW, RW)])


def _tc_first(x_ref, w_ref, deg_ref, hs_ref, dinv_ref):
    deg = jnp.sum(deg_ref[...], axis=1, keepdims=True) + 1.0
    dinv = lax.rsqrt(deg)
    h = jnp.dot(x_ref[...], w_ref[...], preferred_element_type=jnp.float32)
    hs_ref[...] = h * dinv
    dinv_ref[...] = dinv


def _tc_mid(agg_ref, hs1_ref, dinv_ref, b_ref, w_ref, hs2_ref):
    a = agg_ref[...]
    dinv = dinv_ref[...]
    z = (a[0] + a[1] + hs1_ref[...]) * dinv + b_ref[...]
    o = jnp.maximum(z, 0.0)
    hs2_ref[...] = jnp.dot(o, w_ref[...],
                           preferred_element_type=jnp.float32) * dinv


def _tc_last(agg_ref, hs2_ref, dinv_ref, b_ref, fcw_ref, fcb_ref,
             h_ref, s_ref):
    a = agg_ref[...]
    z = (a[0] + a[1] + hs2_ref[...]) * dinv_ref[...] + b_ref[...]
    h = jnp.maximum(z, 0.0)
    h_ref[...] = h
    s_ref[...] = jnp.dot(h, fcw_ref[...],
                         preferred_element_type=jnp.float32) + fcb_ref[...]


_first = pl.pallas_call(
    _tc_first,
    grid=(GRID,),
    in_specs=[
        pl.BlockSpec((BN, D), lambda i: (i, 0)),
        pl.BlockSpec((D, D), lambda i: (0, 0)),
        pl.BlockSpec((BN, NW), lambda i: (i, 0)),
    ],
    out_specs=[
        pl.BlockSpec((BN, D), lambda i: (i, 0)),
        pl.BlockSpec((BN, 1), lambda i: (i, 0)),
    ],
    out_shape=[
        jax.ShapeDtypeStruct((N, D), jnp.float32),
        jax.ShapeDtypeStruct((N, 1), jnp.float32),
    ],
)

_mid = pl.pallas_call(
    _tc_mid,
    grid=(GRID,),
    in_specs=[
        pl.BlockSpec((NC, BN, D), lambda i: (0, i, 0)),
        pl.BlockSpec((BN, D), lambda i: (i, 0)),
        pl.BlockSpec((BN, 1), lambda i: (i, 0)),
        pl.BlockSpec((1, D), lambda i: (0, 0)),
        pl.BlockSpec((D, D), lambda i: (0, 0)),
    ],
    out_specs=pl.BlockSpec((BN, D), lambda i: (i, 0)),
    out_shape=jax.ShapeDtypeStruct((N, D), jnp.float32),
)

_last = pl.pallas_call(
    _tc_last,
    grid=(GRID,),
    in_specs=[
        pl.BlockSpec((NC, BN, D), lambda i: (0, i, 0)),
        pl.BlockSpec((BN, D), lambda i: (i, 0)),
        pl.BlockSpec((BN, 1), lambda i: (i, 0)),
        pl.BlockSpec((1, D), lambda i: (0, 0)),
        pl.BlockSpec((D, 1), lambda i: (0, 0)),
        pl.BlockSpec((1, 1), lambda i: (0, 0)),
    ],
    out_specs=[
        pl.BlockSpec((BN, D), lambda i: (i, 0)),
        pl.BlockSpec((BN, 1), lambda i: (i, 0)),
    ],
    out_shape=[
        jax.ShapeDtypeStruct((N, D), jnp.float32),
        jax.ShapeDtypeStruct((N, 1), jnp.float32),
    ],
)


def kernel(x, edge_index, W1, b1, W2, b2, fc_w, fc_b):
    src3 = edge_index[0].reshape(NW, NBLK, IBLK, CH)
    dst3 = edge_index[1].reshape(NW, NBLK, IBLK, CH)
    zeros_rowsD = jnp.zeros((RW, D), jnp.float32)
    zeros_np = jnp.zeros((NP,), jnp.float32)

    degp = _sc_degree(edge_index[1], zeros_np)
    degp = degp.reshape(NW, NP)[:, :N].T
    hs1, dinv = _first(x, W1, degp)
    agg1 = _sc_edge_agg(hs1, src3, dst3, zeros_rowsD).reshape(NC, NP, D)
    hs2 = _mid(agg1, hs1, dinv, b1.reshape(1, D), W2)
    agg2 = _sc_edge_agg(hs2, src3, dst3, zeros_rowsD).reshape(NC, NP, D)
    h, s = _last(agg2, hs2, dinv, b2.reshape(1, D), fc_w,
                 fc_b.reshape(1, 1))
    return s[:, 0], h


# speedup vs baseline: 2.4552x; 1.1045x over previous
import functools

import jax
import jax.numpy as jnp
from jax import lax
from jax.experimental import pallas as pl
from jax.experimental.pallas import tpu as pltpu
from jax.experimental.pallas import tpu_sc as plsc

N = 10000
E = 320000
D = 128
NC = 2
NS = 16
NW = NC * NS
EW = E // NW
CH = 40
NCHUNK = EW // CH
IBLK = 50
NBLK = NCHUNK // IBLK
NP = 10240
RW = NP // NS
DEGW = 16

BN = 2000
GRID = N // BN

_mesh = plsc.VectorSubcoreMesh(core_axis_name="c", subcore_axis_name="s")


@functools.partial(
    pl.kernel,
    out_type=jax.ShapeDtypeStruct((NW * NP,), jnp.float32),
    mesh=_mesh,
    scratch_types=[
        pltpu.VMEM((EW,), jnp.int32),
        pltpu.VMEM((NP,), jnp.float32),
    ],
    compiler_params=pltpu.CompilerParams(needs_layout_passes=False),
)
def _sc_degree(dst_hbm, zeros_hbm, out_hbm, dstv, degt):
    c = lax.axis_index("c")
    s = lax.axis_index("s")
    wid = c * NS + s
    pltpu.sync_copy(dst_hbm.at[pl.ds(wid * EW, EW)], dstv)
    pltpu.sync_copy(zeros_hbm, degt)
    ones = jnp.ones((16,), jnp.float32)

    def grp(g, carry):
        idx = dstv[pl.ds(g * 16, 16)]
        plsc.addupdate_scatter(degt, [idx], ones)
        return carry

    lax.fori_loop(0, EW // 16, grp, 0)
    pltpu.sync_copy(degt, out_hbm.at[pl.ds(wid * NP, NP)])


@functools.partial(
    pl.kernel,
    out_type=jax.ShapeDtypeStruct((NC * NP, D), jnp.float32),
    mesh=_mesh,
    scratch_types=[
        pltpu.VMEM((IBLK, CH), jnp.int32),
        pltpu.VMEM((IBLK, CH), jnp.int32),
        pltpu.VMEM((CH, D), jnp.float32),
        pltpu.VMEM((CH, D), jnp.float32),
        pltpu.SemaphoreType.DMA,
        pltpu.SemaphoreType.DMA,
        pltpu.VMEM_SHARED((NP, D), jnp.float32),
    ],
)
def _sc_edge_agg(hs_hbm, src_hbm, dst_hbm, zeros_hbm, out_hbm,
                 srcv, dstv, rows0, rows1, sem0, sem1, acc):
    c = lax.axis_index("c")
    s = lax.axis_index("s")
    wid = c * NS + s
    pltpu.sync_copy(zeros_hbm, acc.at[pl.ds(s * RW, RW)])
    plsc.subcore_barrier()

    def block(b, carry):
        pltpu.sync_copy(src_hbm.at[wid].at[b], srcv)
        pltpu.sync_copy(dst_hbm.at[wid].at[b], dstv)
        pltpu.async_copy(hs_hbm.at[srcv.at[0]], rows0, sem0)
        pltpu.async_copy(hs_hbm.at[srcv.at[1]], rows1, sem1)

        def pair(i, c2):
            j0 = 2 * i
            pltpu.make_async_copy(hs_hbm.at[srcv.at[j0]], rows0, sem0).wait()
            pltpu.sync_copy(rows0, acc.at[dstv.at[j0]], add=True)
            pltpu.async_copy(hs_hbm.at[srcv.at[j0 + 2]], rows0, sem0)
            pltpu.make_async_copy(hs_hbm.at[srcv.at[j0 + 1]], rows1,
                                  sem1).wait()
            pltpu.sync_copy(rows1, acc.at[dstv.at[j0 + 1]], add=True)
            pltpu.async_copy(hs_hbm.at[srcv.at[j0 + 3]], rows1, sem1)
            return c2

        lax.fori_loop(0, IBLK // 2 - 1, pair, 0)
        pltpu.make_async_copy(hs_hbm.at[srcv.at[IBLK - 2]], rows0,
                              sem0).wait()
        pltpu.sync_copy(rows0, acc.at[dstv.at[IBLK - 2]], add=True)
        pltpu.make_async_copy(hs_hbm.at[srcv.at[IBLK - 1]], rows1,
                              sem1).wait()
        pltpu.sync_copy(rows1, acc.at[dstv.at[IBLK - 1]], add=True)
        return carry

    lax.fori_loop(0, NBLK, block, 0)
    plsc.subcore_barrier()
    pltpu.sync_copy(acc.at[pl.ds(s * RW, RW)],
                    out_hbm.at[pl.ds(c * NP + s * RW, RW)])


def _tc_first(x_ref, w_ref, deg_ref, hs_ref, dinv_ref):
    deg = jnp.sum(deg_ref[...], axis=1, keepdims=True) + 1.0
    dinv = lax.rsqrt(deg)
    h = jnp.dot(x_ref[...], w_ref[...], preferred_element_type=jnp.float32)
    hs_ref[...] = h * dinv
    dinv_ref[...] = dinv


def _tc_mid(agg_ref, hs1_ref, dinv_ref, b_ref, w_ref, hs2_ref):
    a = agg_ref[...]
    dinv = dinv_ref[...]
    z = (a[0] + a[1] + hs1_ref[...]) * dinv + b_ref[...]
    o = jnp.maximum(z, 0.0)
    hs2_ref[...] = jnp.dot(o, w_ref[...],
                           preferred_element_type=jnp.float32) * dinv


def _tc_last(agg_ref, hs2_ref, dinv_ref, b_ref, fcw_ref, fcb_ref,
             h_ref, s_ref):
    a = agg_ref[...]
    z = (a[0] + a[1] + hs2_ref[...]) * dinv_ref[...] + b_ref[...]
    h = jnp.maximum(z, 0.0)
    h_ref[...] = h
    s_ref[...] = jnp.dot(h, fcw_ref[...],
                         preferred_element_type=jnp.float32) + fcb_ref[...]


_first = pl.pallas_call(
    _tc_first,
    grid=(GRID,),
    in_specs=[
        pl.BlockSpec((BN, D), lambda i: (i, 0)),
        pl.BlockSpec((D, D), lambda i: (0, 0)),
        pl.BlockSpec((BN, NW), lambda i: (i, 0)),
    ],
    out_specs=[
        pl.BlockSpec((BN, D), lambda i: (i, 0)),
        pl.BlockSpec((BN, 1), lambda i: (i, 0)),
    ],
    out_shape=[
        jax.ShapeDtypeStruct((N, D), jnp.float32),
        jax.ShapeDtypeStruct((N, 1), jnp.float32),
    ],
)

_mid = pl.pallas_call(
    _tc_mid,
    grid=(GRID,),
    in_specs=[
        pl.BlockSpec((NC, BN, D), lambda i: (0, i, 0)),
        pl.BlockSpec((BN, D), lambda i: (i, 0)),
        pl.BlockSpec((BN, 1), lambda i: (i, 0)),
        pl.BlockSpec((1, D), lambda i: (0, 0)),
        pl.BlockSpec((D, D), lambda i: (0, 0)),
    ],
    out_specs=pl.BlockSpec((BN, D), lambda i: (i, 0)),
    out_shape=jax.ShapeDtypeStruct((N, D), jnp.float32),
)

_last = pl.pallas_call(
    _tc_last,
    grid=(GRID,),
    in_specs=[
        pl.BlockSpec((NC, BN, D), lambda i: (0, i, 0)),
        pl.BlockSpec((BN, D), lambda i: (i, 0)),
        pl.BlockSpec((BN, 1), lambda i: (i, 0)),
        pl.BlockSpec((1, D), lambda i: (0, 0)),
        pl.BlockSpec((D, 1), lambda i: (0, 0)),
        pl.BlockSpec((1, 1), lambda i: (0, 0)),
    ],
    out_specs=[
        pl.BlockSpec((BN, D), lambda i: (i, 0)),
        pl.BlockSpec((BN, 1), lambda i: (i, 0)),
    ],
    out_shape=[
        jax.ShapeDtypeStruct((N, D), jnp.float32),
        jax.ShapeDtypeStruct((N, 1), jnp.float32),
    ],
)


def kernel(x, edge_index, W1, b1, W2, b2, fc_w, fc_b):
    src3 = edge_index[0].reshape(NW, NBLK, IBLK, CH)
    dst3 = edge_index[1].reshape(NW, NBLK, IBLK, CH)
    zeros_rowsD = jnp.zeros((RW, D), jnp.float32)
    zeros_np = jnp.zeros((NP,), jnp.float32)

    degp = _sc_degree(edge_index[1], zeros_np)
    degp = degp.reshape(NW, NP)[:, :N].T
    hs1, dinv = _first(x, W1, degp)
    agg1 = _sc_edge_agg(hs1, src3, dst3, zeros_rowsD).reshape(NC, NP, D)
    hs2 = _mid(agg1, hs1, dinv, b1.reshape(1, D), W2)
    agg2 = _sc_edge_agg(hs2, src3, dst3, zeros_rowsD).reshape(NC, NP, D)
    h, s = _last(agg2, hs2, dinv, b2.reshape(1, D), fc_w,
                 fc_b.reshape(1, 1))
    return s[:, 0], h
